# Initial kernel scaffold; baseline (speedup 1.0000x reference)
import jax
import jax.numpy as jnp
from jax.experimental import pallas as pl


def kernel(x, edge_index, edge_type, W1, self_w1, b1, W2, self_w2, b2):
    raise NotImplementedError("write your pallas kernel here")


# speedup vs baseline: 3.9349x; 3.9349x over previous
import functools

import jax
import jax.numpy as jnp
from jax import lax
from jax.experimental import pallas as pl
from jax.experimental.pallas import tpu as pltpu
from jax.experimental.pallas import tpu_sc as plsc

NC = 2
NS = 16
NW = NC * NS
LANES = 16
CH = 128
RPW = 656
NPAD = NS * RPW
MBLK = 400


def _mm_body(x_ref, w_ref, sw_ref, y_ref, s_ref):
    x = x_ref[...]
    y_ref[...] = jnp.dot(x, w_ref[...], preferred_element_type=jnp.float32,
                         precision=lax.Precision.HIGHEST)
    s_ref[...] = jnp.dot(x, sw_ref[...], preferred_element_type=jnp.float32,
                         precision=lax.Precision.HIGHEST)


def _mm(h, wf, sw):
    n, d = h.shape
    rh = wf.shape[1]
    hd = sw.shape[1]
    return pl.pallas_call(
        _mm_body,
        grid=(n // MBLK,),
        in_specs=[
            pl.BlockSpec((MBLK, d), lambda i: (i, 0)),
            pl.BlockSpec((d, rh), lambda i: (0, 0)),
            pl.BlockSpec((d, hd), lambda i: (0, 0)),
        ],
        out_specs=[
            pl.BlockSpec((MBLK, rh), lambda i: (i, 0)),
            pl.BlockSpec((MBLK, hd), lambda i: (i, 0)),
        ],
        out_shape=[
            jax.ShapeDtypeStruct((n, rh), jnp.float32),
            jax.ShapeDtypeStruct((n, hd), jnp.float32),
        ],
    )(h, wf, sw)


def _combine_body(relu, a_ref, s_ref, b_ref, o_ref):
    v = a_ref[0] + a_ref[1] + s_ref[...] + b_ref[...]
    o_ref[...] = jnp.maximum(v, 0.0) if relu else v


def _combine(agg, s, b, relu):
    n, h = s.shape
    return pl.pallas_call(
        functools.partial(_combine_body, relu),
        grid=(n // MBLK,),
        in_specs=[
            pl.BlockSpec((2, MBLK, h), lambda i: (0, i, 0)),
            pl.BlockSpec((MBLK, h), lambda i: (i, 0)),
            pl.BlockSpec((1, h), lambda i: (0, 0)),
        ],
        out_specs=pl.BlockSpec((MBLK, h), lambda i: (i, 0)),
        out_shape=jax.ShapeDtypeStruct((n, h), jnp.float32),
    )(agg, s, b.reshape(1, h))


def _sc_agg_call(nchunk, hdim, rel):
    mesh = plsc.VectorSubcoreMesh(core_axis_name="c", subcore_axis_name="s")

    @functools.partial(
        pl.kernel,
        out_type=jax.ShapeDtypeStruct((NC, NPAD, hdim), jnp.float32),
        mesh=mesh,
        scratch_types=[
            pltpu.VMEM((nchunk, CH), jnp.int32),
            pltpu.VMEM((nchunk, CH), jnp.int32),
            pltpu.VMEM((CH, hdim), jnp.float32),
            pltpu.VMEM_SHARED((NPAD, hdim), jnp.float32),
            pltpu.SemaphoreType.DMA,
        ],
    )
    def sc_kernel(src_h, typ_h, dst_h, y_h, out_h, gidx, didx, rows, agg, sem):
        cid = lax.axis_index("c")
        sid = lax.axis_index("s")
        wid = sid * NC + cid

        @pl.loop(0, CH)
        def _(i):
            @pl.loop(0, hdim // LANES)
            def _(j):
                rows[i, pl.ds(j * LANES, LANES)] = jnp.zeros((LANES,), jnp.float32)

        row0 = sid * RPW
        nfull = RPW // CH
        rem = RPW - nfull * CH

        @pl.loop(0, nfull)
        def _(k):
            pltpu.sync_copy(rows, agg.at[pl.ds(row0 + k * CH, CH)])

        pltpu.sync_copy(rows.at[pl.ds(0, rem)],
                        agg.at[pl.ds(row0 + nfull * CH, rem)])

        pltpu.sync_copy(src_h.at[wid], gidx)
        pltpu.sync_copy(typ_h.at[wid], didx)

        @pl.loop(0, nchunk)
        def _(c):
            @pl.loop(0, CH // LANES)
            def _(j):
                s = pl.ds(j * LANES, LANES)
                gidx[c, s] = gidx[c, s] * rel + didx[c, s]

        pltpu.sync_copy(dst_h.at[wid], didx)

        plsc.subcore_barrier()

        @pl.loop(0, nchunk)
        def _(c):
            pltpu.async_copy(y_h.at[gidx.at[c]], rows, sem).wait()
            pltpu.sync_copy(rows, agg.at[didx.at[c]], add=True)

        plsc.subcore_barrier()

        pltpu.sync_copy(agg.at[pl.ds(row0, RPW)],
                        out_h.at[cid, pl.ds(row0, RPW)])

    return sc_kernel


def kernel(x, edge_index, edge_type, W1, self_w1, b1, W2, self_w2, b2):
    n, d = x.shape
    rel, _, hdim = W1.shape
    e = edge_type.shape[0]

    per_w = -(-e // (NW * CH)) * CH
    nchunk = per_w // CH
    epad = NW * per_w - e
    src = jnp.concatenate([edge_index[0], jnp.zeros((epad,), jnp.int32)])
    typ = jnp.concatenate([edge_type, jnp.zeros((epad,), jnp.int32)])
    dst = jnp.concatenate([edge_index[1],
                           jnp.full((epad,), NPAD - 1, jnp.int32)])
    src = src.reshape(NW, nchunk, CH)
    typ = typ.reshape(NW, nchunk, CH)
    dst = dst.reshape(NW, nchunk, CH)

    sc_agg = _sc_agg_call(nchunk, hdim, rel)

    w1f = W1.transpose(1, 0, 2).reshape(d, rel * hdim)
    w2f = W2.transpose(1, 0, 2).reshape(hdim, rel * W2.shape[2])

    y1, s1 = _mm(x, w1f, self_w1)
    a1 = sc_agg(src, typ, dst, y1.reshape(n * rel, hdim))
    h1 = _combine(a1, s1, b1, relu=True)

    y2, s2 = _mm(h1, w2f, self_w2)
    a2 = sc_agg(src, typ, dst, y2.reshape(n * rel, W2.shape[2]))
    return _combine(a2, s2, b2, relu=False)


# speedup vs baseline: 4.1760x; 1.0613x over previous
import functools

import jax
import jax.numpy as jnp
from jax import lax
from jax.experimental import pallas as pl
from jax.experimental.pallas import tpu as pltpu
from jax.experimental.pallas import tpu_sc as plsc

NC = 2
NS = 16
NW = NC * NS
LANES = 16
CH = 128
RPW = 656
NPAD = NS * RPW
MBLK = 400


def _mm_body(x_ref, w_ref, sw_ref, y_ref, s_ref):
    x = x_ref[...]
    y_ref[...] = jnp.dot(x, w_ref[...], preferred_element_type=jnp.float32,
                         precision=lax.Precision.HIGHEST)
    s_ref[...] = jnp.dot(x, sw_ref[...], preferred_element_type=jnp.float32,
                         precision=lax.Precision.HIGHEST)


def _mm(h, wf, sw):
    n, d = h.shape
    rh = wf.shape[1]
    hd = sw.shape[1]
    return pl.pallas_call(
        _mm_body,
        grid=(n // MBLK,),
        in_specs=[
            pl.BlockSpec((MBLK, d), lambda i: (i, 0)),
            pl.BlockSpec((d, rh), lambda i: (0, 0)),
            pl.BlockSpec((d, hd), lambda i: (0, 0)),
        ],
        out_specs=[
            pl.BlockSpec((MBLK, rh), lambda i: (i, 0)),
            pl.BlockSpec((MBLK, hd), lambda i: (i, 0)),
        ],
        out_shape=[
            jax.ShapeDtypeStruct((n, rh), jnp.float32),
            jax.ShapeDtypeStruct((n, hd), jnp.float32),
        ],
    )(h, wf, sw)


def _combine_body(relu, a_ref, s_ref, b_ref, o_ref):
    v = a_ref[0] + a_ref[1] + s_ref[...] + b_ref[...]
    o_ref[...] = jnp.maximum(v, 0.0) if relu else v


def _combine(agg, s, b, relu):
    n, h = s.shape
    return pl.pallas_call(
        functools.partial(_combine_body, relu),
        grid=(n // MBLK,),
        in_specs=[
            pl.BlockSpec((2, MBLK, h), lambda i: (0, i, 0)),
            pl.BlockSpec((MBLK, h), lambda i: (i, 0)),
            pl.BlockSpec((1, h), lambda i: (0, 0)),
        ],
        out_specs=pl.BlockSpec((MBLK, h), lambda i: (i, 0)),
        out_shape=jax.ShapeDtypeStruct((n, h), jnp.float32),
    )(agg, s, b.reshape(1, h))


def _sc_agg_call(nchunk, hdim, rel):
    mesh = plsc.VectorSubcoreMesh(core_axis_name="c", subcore_axis_name="s")

    @functools.partial(
        pl.kernel,
        out_type=jax.ShapeDtypeStruct((NC, NPAD, hdim), jnp.float32),
        mesh=mesh,
        scratch_types=[
            pltpu.VMEM((nchunk, CH), jnp.int32),
            pltpu.VMEM((nchunk, CH), jnp.int32),
            pltpu.VMEM((CH, hdim), jnp.float32),
            pltpu.VMEM((CH, hdim), jnp.float32),
            pltpu.VMEM_SHARED((NPAD, hdim), jnp.float32),
            pltpu.SemaphoreType.DMA,
            pltpu.SemaphoreType.DMA,
        ],
    )
    def sc_kernel(src_h, typ_h, dst_h, y_h, out_h, gidx, didx, rows, rows_b,
                  agg, sem, sem_b):
        cid = lax.axis_index("c")
        sid = lax.axis_index("s")
        wid = sid * NC + cid

        @pl.loop(0, CH)
        def _(i):
            @pl.loop(0, hdim // LANES)
            def _(j):
                rows[i, pl.ds(j * LANES, LANES)] = jnp.zeros((LANES,), jnp.float32)

        row0 = sid * RPW
        nfull = RPW // CH
        rem = RPW - nfull * CH

        @pl.loop(0, nfull)
        def _(k):
            pltpu.sync_copy(rows, agg.at[pl.ds(row0 + k * CH, CH)])

        pltpu.sync_copy(rows.at[pl.ds(0, rem)],
                        agg.at[pl.ds(row0 + nfull * CH, rem)])

        pltpu.sync_copy(src_h.at[wid], gidx)
        pltpu.sync_copy(typ_h.at[wid], didx)

        @pl.loop(0, nchunk)
        def _(c):
            @pl.loop(0, CH // LANES)
            def _(j):
                s = pl.ds(j * LANES, LANES)
                gidx[c, s] = gidx[c, s] * rel + didx[c, s]

        pltpu.sync_copy(dst_h.at[wid], didx)

        pltpu.async_copy(y_h.at[gidx.at[0]], rows, sem)

        plsc.subcore_barrier()

        @pl.loop(0, nchunk, step=2)
        def _(c):
            pltpu.make_async_copy(y_h.at[gidx.at[c]], rows, sem).wait()
            pltpu.async_copy(y_h.at[gidx.at[c + 1]], rows_b, sem_b)
            pltpu.sync_copy(rows, agg.at[didx.at[c]], add=True)
            pltpu.make_async_copy(y_h.at[gidx.at[c + 1]], rows_b, sem_b).wait()

            @pl.when(c + 2 < nchunk)
            def _():
                pltpu.async_copy(y_h.at[gidx.at[c + 2]], rows, sem)

            pltpu.sync_copy(rows_b, agg.at[didx.at[c + 1]], add=True)

        plsc.subcore_barrier()

        pltpu.sync_copy(agg.at[pl.ds(row0, RPW)],
                        out_h.at[cid, pl.ds(row0, RPW)])

    return sc_kernel


def kernel(x, edge_index, edge_type, W1, self_w1, b1, W2, self_w2, b2):
    n, d = x.shape
    rel, _, hdim = W1.shape
    e = edge_type.shape[0]

    per_w = -(-e // (NW * CH)) * CH
    nchunk = per_w // CH
    epad = NW * per_w - e
    src = jnp.concatenate([edge_index[0], jnp.zeros((epad,), jnp.int32)])
    typ = jnp.concatenate([edge_type, jnp.zeros((epad,), jnp.int32)])
    dst = jnp.concatenate([edge_index[1],
                           jnp.full((epad,), NPAD - 1, jnp.int32)])
    src = src.reshape(NW, nchunk, CH)
    typ = typ.reshape(NW, nchunk, CH)
    dst = dst.reshape(NW, nchunk, CH)

    sc_agg = _sc_agg_call(nchunk, hdim, rel)

    w1f = W1.transpose(1, 0, 2).reshape(d, rel * hdim)
    w2f = W2.transpose(1, 0, 2).reshape(hdim, rel * W2.shape[2])

    y1, s1 = _mm(x, w1f, self_w1)
    a1 = sc_agg(src, typ, dst, y1.reshape(n * rel, hdim))
    h1 = _combine(a1, s1, b1, relu=True)

    y2, s2 = _mm(h1, w2f, self_w2)
    a2 = sc_agg(src, typ, dst, y2.reshape(n * rel, W2.shape[2]))
    return _combine(a2, s2, b2, relu=False)


# speedup vs baseline: 4.2525x; 1.0183x over previous
import functools

import jax
import jax.numpy as jnp
from jax import lax
from jax.experimental import pallas as pl
from jax.experimental.pallas import tpu as pltpu
from jax.experimental.pallas import tpu_sc as plsc

NC = 2
NS = 16
NW = NC * NS
LANES = 16
CH = 128
RPW = 656
NPAD = NS * RPW
MBLK = 400


def _mm_body(x_ref, w_ref, sw_ref, y_ref, s_ref):
    x = x_ref[...]
    y_ref[...] = jnp.dot(x, w_ref[...], preferred_element_type=jnp.float32,
                         precision=lax.Precision.HIGHEST)
    s_ref[...] = jnp.dot(x, sw_ref[...], preferred_element_type=jnp.float32,
                         precision=lax.Precision.HIGHEST)


def _mm(h, wf, sw):
    n, d = h.shape
    rh = wf.shape[1]
    hd = sw.shape[1]
    return pl.pallas_call(
        _mm_body,
        grid=(n // MBLK,),
        in_specs=[
            pl.BlockSpec((MBLK, d), lambda i: (i, 0)),
            pl.BlockSpec((d, rh), lambda i: (0, 0)),
            pl.BlockSpec((d, hd), lambda i: (0, 0)),
        ],
        out_specs=[
            pl.BlockSpec((MBLK, rh), lambda i: (i, 0)),
            pl.BlockSpec((MBLK, hd), lambda i: (i, 0)),
        ],
        out_shape=[
            jax.ShapeDtypeStruct((n, rh), jnp.float32),
            jax.ShapeDtypeStruct((n, hd), jnp.float32),
        ],
    )(h, wf, sw)


def _combine_body(relu, a_ref, s_ref, b_ref, o_ref):
    v = a_ref[0] + a_ref[1] + s_ref[...] + b_ref[...]
    o_ref[...] = jnp.maximum(v, 0.0) if relu else v


def _combine(agg, s, b, relu):
    n, h = s.shape
    return pl.pallas_call(
        functools.partial(_combine_body, relu),
        grid=(n // MBLK,),
        in_specs=[
            pl.BlockSpec((2, MBLK, h), lambda i: (0, i, 0)),
            pl.BlockSpec((MBLK, h), lambda i: (i, 0)),
            pl.BlockSpec((1, h), lambda i: (0, 0)),
        ],
        out_specs=pl.BlockSpec((MBLK, h), lambda i: (i, 0)),
        out_shape=jax.ShapeDtypeStruct((n, h), jnp.float32),
    )(agg, s, b.reshape(1, h))


def _sc_agg_call(nchunk, hdim, rel):
    mesh = plsc.VectorSubcoreMesh(core_axis_name="c", subcore_axis_name="s")

    @functools.partial(
        pl.kernel,
        out_type=jax.ShapeDtypeStruct((NC, NPAD, hdim), jnp.float32),
        mesh=mesh,
        scratch_types=[
            pltpu.VMEM((nchunk, CH), jnp.int32),
            pltpu.VMEM((nchunk, CH), jnp.int32),
            pltpu.VMEM((CH, hdim), jnp.float32),
            pltpu.VMEM((CH, hdim), jnp.float32),
            pltpu.VMEM_SHARED((NPAD, hdim), jnp.float32),
            pltpu.SemaphoreType.DMA,
            pltpu.SemaphoreType.DMA,
        ],
    )
    def sc_kernel(src_h, typ_h, dst_h, y_h, out_h, gidx, didx, rows, rows_b,
                  agg, sem, sem_b):
        cid = lax.axis_index("c")
        sid = lax.axis_index("s")
        wid = sid * NC + cid

        @pl.loop(0, CH)
        def _(i):
            @pl.loop(0, hdim // LANES)
            def _(j):
                rows[i, pl.ds(j * LANES, LANES)] = jnp.zeros((LANES,), jnp.float32)

        row0 = sid * RPW
        nfull = RPW // CH
        rem = RPW - nfull * CH

        @pl.loop(0, nfull)
        def _(k):
            pltpu.sync_copy(rows, agg.at[pl.ds(row0 + k * CH, CH)])

        pltpu.sync_copy(rows.at[pl.ds(0, rem)],
                        agg.at[pl.ds(row0 + nfull * CH, rem)])

        pltpu.sync_copy(src_h.at[wid], gidx)
        pltpu.sync_copy(typ_h.at[wid], didx)

        @pl.loop(0, nchunk)
        def _(c):
            @pl.loop(0, CH // LANES)
            def _(j):
                s = pl.ds(j * LANES, LANES)
                gidx[c, s] = gidx[c, s] * rel + didx[c, s]

        pltpu.sync_copy(dst_h.at[wid], didx)

        pltpu.async_copy(y_h.at[gidx.at[0]], rows, sem)

        plsc.subcore_barrier()

        @pl.loop(0, nchunk, step=2)
        def _(c):
            pltpu.async_copy(y_h.at[gidx.at[c + 1]], rows_b, sem_b)
            pltpu.make_async_copy(y_h.at[gidx.at[c]], rows, sem).wait()
            pltpu.sync_copy(rows, agg.at[didx.at[c]], add=True)

            @pl.when(c + 2 < nchunk)
            def _():
                pltpu.async_copy(y_h.at[gidx.at[c + 2]], rows, sem)

            pltpu.make_async_copy(y_h.at[gidx.at[c + 1]], rows_b, sem_b).wait()
            pltpu.sync_copy(rows_b, agg.at[didx.at[c + 1]], add=True)

        plsc.subcore_barrier()

        pltpu.sync_copy(agg.at[pl.ds(row0, RPW)],
                        out_h.at[cid, pl.ds(row0, RPW)])

    return sc_kernel


def kernel(x, edge_index, edge_type, W1, self_w1, b1, W2, self_w2, b2):
    n, d = x.shape
    rel, _, hdim = W1.shape
    e = edge_type.shape[0]

    per_w = -(-e // (NW * CH)) * CH
    nchunk = per_w // CH
    epad = NW * per_w - e
    src = jnp.concatenate([edge_index[0], jnp.zeros((epad,), jnp.int32)])
    typ = jnp.concatenate([edge_type, jnp.zeros((epad,), jnp.int32)])
    dst = jnp.concatenate([edge_index[1],
                           jnp.full((epad,), NPAD - 1, jnp.int32)])
    src = src.reshape(NW, nchunk, CH)
    typ = typ.reshape(NW, nchunk, CH)
    dst = dst.reshape(NW, nchunk, CH)

    sc_agg = _sc_agg_call(nchunk, hdim, rel)

    w1f = W1.transpose(1, 0, 2).reshape(d, rel * hdim)
    w2f = W2.transpose(1, 0, 2).reshape(hdim, rel * W2.shape[2])

    y1, s1 = _mm(x, w1f, self_w1)
    a1 = sc_agg(src, typ, dst, y1.reshape(n * rel, hdim))
    h1 = _combine(a1, s1, b1, relu=True)

    y2, s2 = _mm(h1, w2f, self_w2)
    a2 = sc_agg(src, typ, dst, y2.reshape(n * rel, W2.shape[2]))
    return _combine(a2, s2, b2, relu=False)
